# baseline (speedup 1.0000x reference)
import jax
import jax.numpy as jnp
from jax.experimental import pallas as pl
from jax.experimental.pallas import tpu as pltpu

_G = 32
_RP = 512
_NA = 2


def _mlp_kernel(x_ref, w1_ref, b1_ref, w2_ref, b2_ref, w3_ref, b3_ref, o_ref):
    x = x_ref[...]
    h1 = jnp.dot(x, w1_ref[...], preferred_element_type=jnp.float32)
    h1 = jnp.maximum(h1 + b1_ref[...], 0.0)
    w2 = w2_ref[...]
    b2 = b2_ref[...]
    chunks = []
    for m in range(h1.shape[1] // 256):
        c = jnp.dot(h1[:, 256 * m:256 * (m + 1)], w2,
                    preferred_element_type=jnp.float32)
        chunks.append(jnp.maximum(c + b2, 0.0))
    h2 = jnp.concatenate(chunks, axis=1)
    q = jnp.dot(h2, w3_ref[...], preferred_element_type=jnp.float32)
    o_ref[...] = q + b3_ref[...]


def kernel(x, w1, b1, w2, b2, w3, b3):
    B, S = x.shape
    H = w1.shape[1]
    G = _G
    rows = B // G
    rp = _RP if rows % _RP == 0 else 1

    xp = x.reshape(rows, G * S)
    eye_g = jnp.eye(G, dtype=jnp.float32)
    eye_8 = jnp.eye(8, dtype=jnp.float32)
    w1b = jnp.kron(eye_g, w1)
    b1b = jnp.tile(b1, (1, G))
    w2b = jnp.kron(eye_8, w2)
    b2b = jnp.tile(b2, (1, 8))
    w3s, b3s = w3[:, :_NA], b3[:, :_NA]
    w3b = jnp.kron(eye_g, w3s)
    b3b = jnp.tile(b3s, (1, G))

    fixed = lambda i: (0, 0)
    q = pl.pallas_call(
        _mlp_kernel,
        out_shape=jax.ShapeDtypeStruct((rows, _NA * G), jnp.float32),
        grid=(rows // rp,),
        in_specs=[
            pl.BlockSpec((rp, G * S), lambda i: (i, 0)),
            pl.BlockSpec(w1b.shape, fixed), pl.BlockSpec(b1b.shape, fixed),
            pl.BlockSpec(w2b.shape, fixed), pl.BlockSpec(b2b.shape, fixed),
            pl.BlockSpec(w3b.shape, fixed), pl.BlockSpec(b3b.shape, fixed),
        ],
        out_specs=pl.BlockSpec((rp, _NA * G), lambda i: (i, 0)),
        compiler_params=pltpu.CompilerParams(
            dimension_semantics=("parallel",)),
    )(xp, w1b, b1b, w2b, b2b, w3b, b3b)
    return q.reshape(B, _NA)


# speedup vs baseline: 2.6626x; 2.6626x over previous
import jax
import jax.numpy as jnp
from jax.experimental import pallas as pl
from jax.experimental.pallas import tpu as pltpu

_R = 16384
_NA = 2

_CN = (((0,), (1,)), ((), ()))
_CC = (((0,), (0,)), ((), ()))


def _mlp_kernel(x_ref, w1_ref, b1t_ref, w2_ref, b2t_ref, w3a_ref, o_ref):
    x = x_ref[...]
    h1 = jax.lax.dot_general(w1_ref[...], x, _CN,
                             preferred_element_type=jnp.float32)
    h1 = jnp.maximum(h1 + b1t_ref[...], 0.0)
    h2 = jax.lax.dot_general(w2_ref[...], h1, _CC,
                             preferred_element_type=jnp.float32)
    h2 = jnp.maximum(h2 + b2t_ref[...], 0.0)
    ones = jnp.ones((1, h2.shape[1]), jnp.float32)
    h2a = jnp.concatenate([h2, ones], axis=0)
    q = jax.lax.dot_general(h2a, w3a_ref[...], _CC,
                            preferred_element_type=jnp.float32)
    o_ref[...] = q


def kernel(x, w1, b1, w2, b2, w3, b3):
    B, S = x.shape
    r = _R if B % _R == 0 else B
    b1t = b1.T
    b2t = b2.T
    w3a = jnp.concatenate([w3[:, :_NA], b3[:, :_NA]], axis=0)

    fixed = lambda i: (0, 0)
    return pl.pallas_call(
        _mlp_kernel,
        out_shape=jax.ShapeDtypeStruct((B, _NA), jnp.float32),
        grid=(B // r,),
        in_specs=[
            pl.BlockSpec((r, S), lambda i: (i, 0)),
            pl.BlockSpec(w1.shape, fixed), pl.BlockSpec(b1t.shape, fixed),
            pl.BlockSpec(w2.shape, fixed), pl.BlockSpec(b2t.shape, fixed),
            pl.BlockSpec(w3a.shape, fixed),
        ],
        out_specs=pl.BlockSpec((r, _NA), lambda i: (i, 0)),
        compiler_params=pltpu.CompilerParams(
            dimension_semantics=("parallel",)),
    )(x, w1, b1t, w2, b2t, w3a)


# speedup vs baseline: 5.3068x; 1.9931x over previous
import jax
import jax.numpy as jnp
from jax.experimental import pallas as pl
from jax.experimental.pallas import tpu as pltpu

_R = 16384
_NA = 2

_CN = (((0,), (1,)), ((), ()))
_CC = (((0,), (0,)), ((), ()))


def _mlp_kernel(x_ref, w1_ref, b1t_ref, w2_ref, b2t_ref, w3a_ref, o_ref):
    x = x_ref[...]
    h1 = jax.lax.dot_general(w1_ref[...], x, _CN,
                             preferred_element_type=jnp.float32)
    h1 = jnp.maximum(h1 + b1t_ref[...], 0.0)
    h2 = jax.lax.dot_general(w2_ref[...], h1, _CC,
                             preferred_element_type=jnp.float32)
    h2 = jnp.maximum(h2 + b2t_ref[...], 0.0)
    ones = jnp.ones((1, h2.shape[1]), jnp.float32)
    h2a = jnp.concatenate([h2, ones], axis=0)
    q = jax.lax.dot_general(w3a_ref[...], h2a, _CC,
                            preferred_element_type=jnp.float32)
    o_ref[...] = q


def kernel(x, w1, b1, w2, b2, w3, b3):
    B, S = x.shape
    r = _R if B % _R == 0 else B
    b1t = b1.T
    b2t = b2.T
    w3a = jnp.concatenate([w3[:, :_NA], b3[:, :_NA]], axis=0)

    fixed = lambda i: (0, 0)
    qt = pl.pallas_call(
        _mlp_kernel,
        out_shape=jax.ShapeDtypeStruct((_NA, B), jnp.float32),
        grid=(B // r,),
        in_specs=[
            pl.BlockSpec((r, S), lambda i: (i, 0)),
            pl.BlockSpec(w1.shape, fixed), pl.BlockSpec(b1t.shape, fixed),
            pl.BlockSpec(w2.shape, fixed), pl.BlockSpec(b2t.shape, fixed),
            pl.BlockSpec(w3a.shape, fixed),
        ],
        out_specs=pl.BlockSpec((_NA, r), lambda i: (0, i)),
        compiler_params=pltpu.CompilerParams(
            dimension_semantics=("parallel",)),
    )(x, w1, b1t, w2, b2t, w3a)
    return qt.T
